# baseline (device time: 71308 ns/iter reference)
import numpy as np

import jax
import jax.numpy as jnp
from jax import lax
from jax.experimental import pallas as pl
from jax.experimental.pallas import tpu as pltpu

N_DEV = 4
B_PER = 2
SQ = 512
SKV = 512
HQ = 32
HQ_PER = 8
H_HALF = 4
DH = 64
D_MODEL = 768
BLK = 64
HALF = H_HALF * DH

_R0_BLOCKS = [0, 3, 6]
_R12_BLOCKS = [1, 2, 4, 5, 7]
_C12_BLOCKS = [0, 1, 2, 4, 5, 7]
_ROW_ORDER = _R0_BLOCKS + _R12_BLOCKS
N0 = len(_R0_BLOCKS) * BLK
N12 = len(_R12_BLOCKS) * BLK
C12 = len(_C12_BLOCKS) * BLK

_ROW_PERM = np.concatenate([np.arange(b * BLK, (b + 1) * BLK) for b in _ROW_ORDER])
_C12_IDX = np.concatenate([np.arange(b * BLK, (b + 1) * BLK) for b in _C12_BLOCKS])
_R0_IDX = _ROW_PERM[:N0]


def _keep(qb, kb):
    return (qb == kb) or (kb == 0) or ((qb + kb) % 3 == 0)


_MASK12 = np.zeros((N12, C12), np.float32)
for _ri, _qb in enumerate(_R12_BLOCKS):
    for _ci, _kb in enumerate(_C12_BLOCKS):
        if _keep(_qb, _kb):
            _MASK12[_ri * BLK:(_ri + 1) * BLK, _ci * BLK:(_ci + 1) * BLK] = 1.0


def kernel(x, Wq, K_ext, V_ext, Wo):
    my = lax.axis_index("i")

    Ks = lax.dynamic_slice_in_dim(K_ext, my * B_PER, B_PER, axis=0)
    Vs = lax.dynamic_slice_in_dim(V_ext, my * B_PER, B_PER, axis=0)

    def slab(t, blocks):
        return jnp.concatenate(
            [
                jnp.transpose(t[:, b * BLK:(b + 1) * BLK], (0, 2, 1, 3))
                for b in blocks
            ],
            axis=2,
        ).astype(jnp.bfloat16)

    k0 = slab(Ks, _R0_BLOCKS)
    v0 = slab(Vs, _R0_BLOCKS)
    k12 = slab(Ks, _C12_BLOCKS)
    v12 = slab(Vs, _C12_BLOCKS)

    xb = jnp.concatenate(
        [x[:, b * BLK:(b + 1) * BLK, :] for b in _ROW_ORDER], axis=1
    ).astype(jnp.bfloat16)
    wqb = Wq.astype(jnp.bfloat16)
    wob = Wo.astype(jnp.bfloat16)
    m12 = jnp.asarray(_MASK12)

    def body(x_ref, wq_ref, k0_ref, v0_ref, k12_ref, v12_ref, m12_ref,
             wo_ref, out_ref,
             wqA, woA, wqB, woB, ctx_ref, acc_ref,
             qA_send, qA_recv, oA_send, oA_recv,
             qB_send, qB_recv, oB_send, oB_recv):
        me = lax.axis_index("i")
        left = lax.rem(me + N_DEV - 1, N_DEV)
        right = lax.rem(me + 1, N_DEV)

        wqA[0] = wq_ref[:, :HALF]
        wqB[0] = wq_ref[:, HALF:]
        woA[0] = wo_ref[:HALF, :]
        woB[0] = wo_ref[HALF:, :]

        barrier_sem = pltpu.get_barrier_semaphore()
        for nbr in (left, right):
            pl.semaphore_signal(
                barrier_sem, inc=1,
                device_id=(nbr,), device_id_type=pl.DeviceIdType.MESH,
            )
        pl.semaphore_wait(barrier_sem, 2)

        for hop in range(N_DEV):
            s = hop % 2
            r = (hop + 1) % 2
            rdmas = []
            if hop < N_DEV - 1:
                for buf, ssem, rsem, dest in (
                    (wqA, qA_send, qA_recv, right),
                    (woA, oA_send, oA_recv, right),
                    (wqB, qB_send, qB_recv, left),
                    (woB, oB_send, oB_recv, left),
                ):
                    rd = pltpu.make_async_remote_copy(
                        src_ref=buf.at[s], dst_ref=buf.at[r],
                        send_sem=ssem.at[s], recv_sem=rsem.at[r],
                        device_id=(dest,), device_id_type=pl.DeviceIdType.MESH,
                    )
                    rd.start()
                    rdmas.append(rd)

            jA = lax.rem(me - hop + N_DEV, N_DEV)
            jB = lax.rem(me + hop, N_DEV)
            for b in range(B_PER):
                acc = None
                for wqbuf, wobuf, j, hbase in (
                    (wqA, woA, jA, 0),
                    (wqB, woB, jB, H_HALF),
                ):
                    qg = (lax.dot_general(
                        x_ref[b], wqbuf[s], (((1,), (0,)), ((), ())),
                        preferred_element_type=jnp.float32,
                    ) * 0.125).astype(jnp.bfloat16)
                    for hh in range(H_HALF):
                        head = j * HQ_PER + hbase + hh
                        c0 = (hbase + hh) * DH
                        q0 = qg[:N0, hh * DH:(hh + 1) * DH]
                        s0 = lax.dot_general(
                            q0, k0_ref[b, head], (((1,), (1,)), ((), ())),
                            preferred_element_type=jnp.float32,
                        )
                        e0 = jnp.exp(s0)
                        den0 = jnp.sum(e0, axis=1, keepdims=True)
                        ctx0 = lax.dot_general(
                            e0.astype(jnp.bfloat16), v0_ref[b, head],
                            (((1,), (0,)), ((), ())),
                            preferred_element_type=jnp.float32,
                        ) * (1.0 / den0)
                        ctx_ref[:N0, c0:c0 + DH] = ctx0.astype(jnp.bfloat16)
                        q12 = qg[N0:, hh * DH:(hh + 1) * DH]
                        s12 = lax.dot_general(
                            q12, k12_ref[b, head], (((1,), (1,)), ((), ())),
                            preferred_element_type=jnp.float32,
                        )
                        e12 = jnp.exp(s12) * m12_ref[...]
                        den12 = jnp.sum(e12, axis=1, keepdims=True)
                        ctx12 = lax.dot_general(
                            e12.astype(jnp.bfloat16), v12_ref[b, head],
                            (((1,), (0,)), ((), ())),
                            preferred_element_type=jnp.float32,
                        ) * (1.0 / den12)
                        ctx_ref[N0:, c0:c0 + DH] = ctx12.astype(jnp.bfloat16)
                    part = lax.dot_general(
                        ctx_ref[:, hbase * DH:(hbase + H_HALF) * DH],
                        wobuf[s], (((1,), (0,)), ((), ())),
                        preferred_element_type=jnp.float32,
                    )
                    acc = part if acc is None else acc + part
                if hop == 0:
                    acc_ref[b] = acc
                else:
                    acc_ref[b] = acc_ref[b] + acc

            for rd in rdmas:
                rd.wait()

        for b in range(B_PER):
            for pos, ob in enumerate(_ROW_ORDER):
                out_ref[b, ob * BLK:(ob + 1) * BLK, :] = (
                    acc_ref[b, pos * BLK:(pos + 1) * BLK, :]
                )

    return pl.pallas_call(
        body,
        out_shape=jax.ShapeDtypeStruct((B_PER, SQ, D_MODEL), jnp.float32),
        in_specs=[pl.BlockSpec(memory_space=pltpu.VMEM)] * 8,
        out_specs=pl.BlockSpec(memory_space=pltpu.VMEM),
        scratch_shapes=[
            pltpu.VMEM((2, D_MODEL, HALF), jnp.bfloat16),
            pltpu.VMEM((2, HALF, D_MODEL), jnp.bfloat16),
            pltpu.VMEM((2, D_MODEL, HALF), jnp.bfloat16),
            pltpu.VMEM((2, HALF, D_MODEL), jnp.bfloat16),
            pltpu.VMEM((SQ, HQ_PER * DH), jnp.bfloat16),
            pltpu.VMEM((B_PER, SQ, D_MODEL), jnp.float32),
            pltpu.SemaphoreType.DMA((2,)),
            pltpu.SemaphoreType.DMA((2,)),
            pltpu.SemaphoreType.DMA((2,)),
            pltpu.SemaphoreType.DMA((2,)),
            pltpu.SemaphoreType.DMA((2,)),
            pltpu.SemaphoreType.DMA((2,)),
            pltpu.SemaphoreType.DMA((2,)),
            pltpu.SemaphoreType.DMA((2,)),
        ],
        compiler_params=pltpu.CompilerParams(collective_id=0),
    )(xb, wqb, k0, v0, k12, v12, m12, wob)


# device time: 66575 ns/iter; 1.0711x vs baseline; 1.0711x over previous
import jax
import jax.numpy as jnp
from jax import lax
from jax.experimental import pallas as pl
from jax.experimental.pallas import tpu as pltpu

N_DEV = 4
B_PER = 2
SQ = 512
SKV = 512
HQ_PER = 8
H_HALF = 4
DH = 64
D_MODEL = 768
BLK = 64
HALF = H_HALF * DH


def kernel(x, Wq, K_ext, V_ext, Wo):
    my = lax.axis_index("i")

    Ks = lax.dynamic_slice_in_dim(K_ext, my * B_PER, B_PER, axis=0)
    Vs = lax.dynamic_slice_in_dim(V_ext, my * B_PER, B_PER, axis=0)
    k4 = jnp.transpose(Ks, (0, 2, 1, 3)).astype(jnp.bfloat16)
    v4 = jnp.transpose(Vs, (0, 2, 1, 3)).astype(jnp.bfloat16)
    xb = (x * 0.125).astype(jnp.bfloat16)
    wqb = Wq.astype(jnp.bfloat16)
    wob = Wo.astype(jnp.bfloat16)

    def body(x_ref, wq_ref, k_ref, v_ref, wo_ref, out_ref,
             wqA, woA, wqB, woB, mask_ref, ctx_ref,
             qA_send, qA_recv, oA_send, oA_recv,
             qB_send, qB_recv, oB_send, oB_recv):
        me = lax.axis_index("i")
        left = lax.rem(me + N_DEV - 1, N_DEV)
        right = lax.rem(me + 1, N_DEV)

        qi = lax.broadcasted_iota(jnp.int32, (SQ, SKV), 0) // BLK
        ki = lax.broadcasted_iota(jnp.int32, (SQ, SKV), 1) // BLK
        keep = (qi == ki) | (ki == 0) | (((qi + ki) % 3) == 0)
        mask_ref[...] = keep.astype(jnp.float32)
        wqA[0] = wq_ref[:, :HALF]
        wqB[0] = wq_ref[:, HALF:]
        woA[0] = wo_ref[:HALF, :]
        woB[0] = wo_ref[HALF:, :]

        barrier_sem = pltpu.get_barrier_semaphore()
        for nbr in (left, right):
            pl.semaphore_signal(
                barrier_sem, inc=1,
                device_id=(nbr,), device_id_type=pl.DeviceIdType.MESH,
            )
        pl.semaphore_wait(barrier_sem, 2)

        for hop in range(N_DEV):
            s = hop % 2
            r = (hop + 1) % 2
            rdmas = []
            if hop < N_DEV - 1:
                for buf, ssem, rsem, dest in (
                    (wqA, qA_send, qA_recv, right),
                    (woA, oA_send, oA_recv, right),
                    (wqB, qB_send, qB_recv, left),
                    (woB, oB_send, oB_recv, left),
                ):
                    rd = pltpu.make_async_remote_copy(
                        src_ref=buf.at[s], dst_ref=buf.at[r],
                        send_sem=ssem.at[s], recv_sem=rsem.at[r],
                        device_id=(dest,), device_id_type=pl.DeviceIdType.MESH,
                    )
                    rd.start()
                    rdmas.append(rd)

            jA = lax.rem(me - hop + N_DEV, N_DEV)
            jB = lax.rem(me + hop, N_DEV)
            for b in range(B_PER):
                acc = None
                for wqbuf, wobuf, j, hbase in (
                    (wqA, woA, jA, 0),
                    (wqB, woB, jB, H_HALF),
                ):
                    qg = lax.dot_general(
                        x_ref[b], wqbuf[s], (((1,), (0,)), ((), ())),
                        preferred_element_type=jnp.float32,
                    ).astype(jnp.bfloat16)
                    for hh in range(H_HALF):
                        head = j * HQ_PER + hbase + hh
                        q1 = qg[:, hh * DH:(hh + 1) * DH]
                        kk = k_ref[b, head]
                        vv = v_ref[b, head]
                        sc = lax.dot_general(
                            q1, kk, (((1,), (1,)), ((), ())),
                            preferred_element_type=jnp.float32,
                        )
                        e = jnp.exp(sc) * mask_ref[...]
                        den = jnp.sum(e, axis=1, keepdims=True)
                        ctx = lax.dot_general(
                            e.astype(jnp.bfloat16), vv, (((1,), (0,)), ((), ())),
                            preferred_element_type=jnp.float32,
                        ) * (1.0 / den)
                        c0 = (hbase + hh) * DH
                        ctx_ref[:, c0:c0 + DH] = ctx.astype(jnp.bfloat16)
                    part = lax.dot_general(
                        ctx_ref[:, hbase * DH:(hbase + H_HALF) * DH],
                        wobuf[s], (((1,), (0,)), ((), ())),
                        preferred_element_type=jnp.float32,
                    )
                    acc = part if acc is None else acc + part
                if hop == 0:
                    out_ref[b] = acc
                else:
                    out_ref[b] = out_ref[b] + acc

            for rd in rdmas:
                rd.wait()

    return pl.pallas_call(
        body,
        out_shape=jax.ShapeDtypeStruct((B_PER, SQ, D_MODEL), jnp.float32),
        in_specs=[pl.BlockSpec(memory_space=pltpu.VMEM)] * 5,
        out_specs=pl.BlockSpec(memory_space=pltpu.VMEM),
        scratch_shapes=[
            pltpu.VMEM((2, D_MODEL, HALF), jnp.bfloat16),
            pltpu.VMEM((2, HALF, D_MODEL), jnp.bfloat16),
            pltpu.VMEM((2, D_MODEL, HALF), jnp.bfloat16),
            pltpu.VMEM((2, HALF, D_MODEL), jnp.bfloat16),
            pltpu.VMEM((SQ, SKV), jnp.float32),
            pltpu.VMEM((SQ, HQ_PER * DH), jnp.bfloat16),
            pltpu.SemaphoreType.DMA((2,)),
            pltpu.SemaphoreType.DMA((2,)),
            pltpu.SemaphoreType.DMA((2,)),
            pltpu.SemaphoreType.DMA((2,)),
            pltpu.SemaphoreType.DMA((2,)),
            pltpu.SemaphoreType.DMA((2,)),
            pltpu.SemaphoreType.DMA((2,)),
            pltpu.SemaphoreType.DMA((2,)),
        ],
        compiler_params=pltpu.CompilerParams(collective_id=0),
    )(xb, wqb, k4, v4, wob)
